# baseline (speedup 1.0000x reference)
import math

import jax
import jax.numpy as jnp
import numpy as np
from jax.experimental import pallas as pl

_NUM_CLASSES = 20
_CUTOFF = 1.0

_V = [math.exp(-(i - 2) ** 2 / 2.0) for i in range(5)]
_V = [x / sum(_V) for x in _V]

_BLK = 256
_SLC = 384


def _row_band_matrix():
    import ml_dtypes
    v32 = np.array(_V, np.float32)
    vhi = v32.astype(ml_dtypes.bfloat16)
    vlo = (v32 - vhi.astype(np.float32)).astype(ml_dtypes.bfloat16)
    n = np.zeros((3 * _SLC, 3 * _BLK), np.float32)
    r = np.arange(_SLC)[:, None]
    w = np.arange(_BLK)[None, :]
    d = r - w
    band = (d >= 0) & (d <= 4)
    dd = np.where(band, d, 0)
    box = band.astype(np.float32)
    ghi = np.where(band, np.asarray(vhi, np.float32)[dd], 0.0)
    glo = np.where(band, np.asarray(vlo, np.float32)[dd], 0.0)
    for t in range(3):
        n[t * _SLC:(t + 1) * _SLC, 0:_BLK] = box
        n[t * _SLC:(t + 1) * _SLC, _BLK:2 * _BLK] = ghi
        n[t * _SLC:(t + 1) * _SLC, 2 * _BLK:3 * _BLK] = glo
    return jnp.asarray(n).astype(jnp.bfloat16)


def _dot(m, x):
    return jax.lax.dot_general(
        m, x, (((1,), (0,)), ((), ())),
        precision=jax.lax.Precision.HIGHEST,
        preferred_element_type=jnp.float32)


def _body(dp_ref, lp_ref, n3_ref, out_ref):
    H, W = out_ref.shape[1], out_ref.shape[2]
    dp = dp_ref[0]
    lp = lp_ref[0]
    n3 = n3_ref[...]

    JH, JW = H + 4, W + 128
    nblk = W // _BLK
    base = dp[2:2 + JH, 2:2 + JW]

    cols = jax.lax.broadcasted_iota(jnp.int32, (1, JW), 1)
    colmask = jnp.where((cols >= 2) & (cols < W + 2), 1.0, 0.0)

    hh = jax.lax.broadcasted_iota(jnp.int32, (H, JH), 0)
    rr = jax.lax.broadcasted_iota(jnp.int32, (H, JH), 1)
    dlt = rr - hh
    rowok = (rr >= 2) & (rr < JH - 2)
    mb = jnp.where((dlt >= 0) & (dlt <= 4) & rowok, 1.0, 0.0)
    mg = jnp.zeros((H, JH), jnp.float32)
    for i in range(5):
        mg = jnp.where((dlt == i) & rowok, _V[i], mg)

    slots = []

    for k in range(25):
        if k == 12:
            continue
        dh, dw = k // 5 - 2, k % 5 - 2
        lab = lp[2 + dh:2 + dh + H, 2 + dw:2 + dw + W]
        nb = dp[2 + dh:2 + dh + JH, 2 + dw:2 + dw + JW]
        jp = jnp.abs(nb - base) * colmask
        a = jp.astype(jnp.bfloat16)
        r1 = jp - a.astype(jnp.float32)
        b = r1.astype(jnp.bfloat16)
        c = (r1 - b.astype(jnp.float32)).astype(jnp.bfloat16)
        rbs, rgs = [], []
        for blk in range(nblk):
            sl = slice(_BLK * blk, _BLK * blk + _SLC)
            j3 = jnp.concatenate([a[:, sl], b[:, sl], c[:, sl]], axis=1)
            o = jax.lax.dot_general(
                j3, n3, (((1,), (0,)), ((), ())),
                preferred_element_type=jnp.float32)
            rbs.append(o[:, 0:_BLK])
            rgs.append(o[:, _BLK:2 * _BLK] + o[:, 2 * _BLK:3 * _BLK])
        rb = jnp.concatenate(rbs, axis=1)
        rg = jnp.concatenate(rgs, axis=1)
        dist = _dot(mb, rb) - _dot(mg, rg)

        key = (jax.lax.bitcast_convert_type(dist, jnp.int32)
               & jnp.int32(-32)) | lab
        if len(slots) < 4:
            slots.append(key)
        else:
            ck = key
            for i in range(4):
                nk = jnp.minimum(slots[i], ck)
                ck = jnp.maximum(slots[i], ck)
                slots[i] = nk

    cut = jnp.int32(0x3F800000)
    ls = [lp[2:2 + H, 2:2 + W]]
    for i in range(4):
        di = slots[i] & jnp.int32(-32)
        ls.append(jnp.where(di > cut, _NUM_CLASSES, slots[i] & 31))

    ones = jnp.ones_like(ls[0])
    cnt = [ones, ones, ones, ones, ones]
    for i in range(5):
        for j in range(i + 1, 5):
            e = jnp.where(ls[i] == ls[j], 1, 0)
            cnt[i] = cnt[i] + e
            cnt[j] = cnt[j] + e
    key = jnp.zeros_like(ls[0])
    for i in range(5):
        ki = jnp.where(ls[i] == _NUM_CLASSES, 0,
                       cnt[i] * 32 + (31 - ls[i]))
        key = jnp.maximum(key, ki)
    best = jnp.where(key > 0, 31 - (key & 31), 0)
    out_ref[0] = best


def kernel(depth, label):
    B, C, H, W = depth.shape
    d = depth[:, 0]
    dp = jnp.pad(d, ((0, 0), (4, 4), (4, 128)))
    lp = jnp.pad(label, ((0, 0), (2, 2), (2, 2)))
    n3 = _row_band_matrix()
    return pl.pallas_call(
        _body,
        grid=(B,),
        in_specs=[
            pl.BlockSpec((1, H + 8, W + 132), lambda b: (b, 0, 0)),
            pl.BlockSpec((1, H + 4, W + 4), lambda b: (b, 0, 0)),
            pl.BlockSpec((3 * _SLC, 3 * _BLK), lambda b: (0, 0)),
        ],
        out_specs=pl.BlockSpec((1, H, W), lambda b: (b, 0, 0)),
        out_shape=jax.ShapeDtypeStruct((B, H, W), jnp.int32),
    )(dp, lp, n3)


# speedup vs baseline: 1.4557x; 1.4557x over previous
import math

import jax
import jax.numpy as jnp
from jax.experimental import pallas as pl

_NUM_CLASSES = 20
_CUTOFF = 1.0

_V = [math.exp(-(i - 2) ** 2 / 2.0) for i in range(5)]
_V = [x / sum(_V) for x in _V]


def _dot(m, x):
    return jax.lax.dot_general(
        m, x, (((1,), (0,)), ((), ())),
        precision=jax.lax.Precision.HIGHEST,
        preferred_element_type=jnp.float32)


def _body(dp_ref, lp_ref, out_ref):
    H, W = out_ref.shape[1], out_ref.shape[2]
    dp = dp_ref[0]
    lp = lp_ref[0]
    JH = H + 4

    S = [dp[:, 4 + j:4 + j + W] for j in range(-4, 5)]
    B = [S[4 + a][2:2 + JH] for a in range(-2, 3)]

    cols = jax.lax.broadcasted_iota(jnp.int32, (1, W), 1)
    CM = [jnp.where((cols + a >= 0) & (cols + a <= W - 1), 1.0, 0.0)
          for a in range(-2, 3)]

    hh = jax.lax.broadcasted_iota(jnp.int32, (H, JH), 0)
    rr = jax.lax.broadcasted_iota(jnp.int32, (H, JH), 1)
    dlt = rr - hh
    rowok = (rr >= 2) & (rr < JH - 2)
    mb = jnp.where((dlt >= 0) & (dlt <= 4) & rowok, 1.0, 0.0)
    mg = jnp.zeros((H, JH), jnp.float32)
    for i in range(5):
        mg = jnp.where((dlt == i) & rowok, _V[i], mg)

    slots = []

    for dh in range(-2, 3):
        T = [S[4 + j][2 + dh:2 + dh + JH] for j in range(-4, 5)]
        for dw in range(-2, 3):
            if dh == 0 and dw == 0:
                continue
            t = [jnp.abs(T[4 + dw + a] - B[2 + a]) * CM[2 + a]
                 for a in range(-2, 3)]
            u0 = t[0] + t[4]
            u1 = t[1] + t[3]
            rb = (u0 + u1) + t[2]
            rg = _V[0] * u0 + (_V[1] * u1 + _V[2] * t[2])
            dist = _dot(mb, rb) - _dot(mg, rg)

            lab = lp[2 + dh:2 + dh + H, 2 + dw:2 + dw + W]
            key = (jax.lax.bitcast_convert_type(dist, jnp.int32)
                   & jnp.int32(-32)) | lab
            if len(slots) < 4:
                slots.append(key)
            else:
                ck = key
                for i in range(4):
                    nk = jnp.minimum(slots[i], ck)
                    ck = jnp.maximum(slots[i], ck)
                    slots[i] = nk

    cut = jnp.int32(0x3F800000)
    ls = [lp[2:2 + H, 2:2 + W]]
    for i in range(4):
        di = slots[i] & jnp.int32(-32)
        ls.append(jnp.where(di > cut, _NUM_CLASSES, slots[i] & 31))

    ones = jnp.ones_like(ls[0])
    cnt = [ones, ones, ones, ones, ones]
    for i in range(5):
        for j in range(i + 1, 5):
            e = jnp.where(ls[i] == ls[j], 1, 0)
            cnt[i] = cnt[i] + e
            cnt[j] = cnt[j] + e
    key = jnp.zeros_like(ls[0])
    for i in range(5):
        ki = jnp.where(ls[i] == _NUM_CLASSES, 0,
                       cnt[i] * 32 + (31 - ls[i]))
        key = jnp.maximum(key, ki)
    best = jnp.where(key > 0, 31 - (key & 31), 0)
    out_ref[0] = best


def kernel(depth, label):
    B, C, H, W = depth.shape
    d = depth[:, 0]
    dp = jnp.pad(d, ((0, 0), (4, 4), (4, 4)))
    lp = jnp.pad(label, ((0, 0), (2, 2), (2, 2)))
    return pl.pallas_call(
        _body,
        grid=(B,),
        in_specs=[
            pl.BlockSpec((1, H + 8, W + 8), lambda b: (b, 0, 0)),
            pl.BlockSpec((1, H + 4, W + 4), lambda b: (b, 0, 0)),
        ],
        out_specs=pl.BlockSpec((1, H, W), lambda b: (b, 0, 0)),
        out_shape=jax.ShapeDtypeStruct((B, H, W), jnp.int32),
    )(dp, lp)
